# Initial kernel scaffold; baseline (speedup 1.0000x reference)
import jax
import jax.numpy as jnp
from jax.experimental import pallas as pl


def kernel(query_X, key_X, value_X, Wq, bq, Wk, bk, Wv, bv):
    raise NotImplementedError("write your pallas kernel here")


# speedup vs baseline: 49.8352x; 49.8352x over previous
import functools
import itertools

import jax
import jax.numpy as jnp
import numpy as np
from jax.experimental import pallas as pl


@functools.lru_cache(maxsize=None)
def _connection_mask(board_size):
    dims = len(board_size)
    dirs = [d for d in itertools.product((-1, 0, 1), repeat=dims)
            if any(x != 0 for x in d)]
    strides = []
    s = 1
    for D in reversed(board_size):
        strides.append(s)
        s *= D
    strides = strides[::-1]
    N = s
    mask = np.zeros((N, N), dtype=np.uint8)
    for fi, idx in enumerate(itertools.product(*(range(D) for D in board_size))):
        for d in dirs:
            t = 1
            while True:
                n = tuple(i + t * di for i, di in zip(idx, d))
                if all(0 <= j < D for j, D in zip(n, board_size)):
                    mask[fi, sum(j * st for j, st in zip(n, strides))] = 1
                    t += 1
                else:
                    break
    return mask


def _attn_kernel(xq_ref, xk_ref, xv_ref, wq_ref, bq_ref, wk_ref, bk_ref,
                 wv_ref, bv_ref, mask_ref, out_ref, *, scale):
    q = jax.lax.dot(xq_ref[0], wq_ref[...],
                    preferred_element_type=jnp.float32) + bq_ref[...]
    k = jax.lax.dot(xk_ref[0], wk_ref[...],
                    preferred_element_type=jnp.float32) + bk_ref[...]
    v = jax.lax.dot(xv_ref[0], wv_ref[...],
                    preferred_element_type=jnp.float32) + bv_ref[...]
    s = jax.lax.dot_general(q, k, (((1,), (1,)), ((), ())),
                            preferred_element_type=jnp.float32) * scale
    s = jnp.where(mask_ref[...] != 0, s, -1e30)
    m = jnp.max(s, axis=1, keepdims=True)
    e = jnp.exp(s - m)
    denom = jnp.sum(e, axis=1, keepdims=True)
    att = e / denom
    out_ref[0] = jax.lax.dot(att, v, preferred_element_type=jnp.float32)


def kernel(query_X, key_X, value_X, Wq, bq, Wk, bk, Wv, bv):
    B = query_X.shape[0]
    board = tuple(int(d) for d in query_X.shape[1:-1])
    in_dim = query_X.shape[-1]
    cmp_dim = Wq.shape[1]
    out_dim = Wv.shape[1]
    mask = jnp.asarray(_connection_mask(board))
    N = mask.shape[0]

    xq = query_X.reshape(B, N, in_dim)
    xk = key_X.reshape(B, N, in_dim)
    xv = value_X.reshape(B, N, in_dim)

    row = lambda b: (b, 0, 0)
    fixed2 = lambda b: (0, 0)
    grid_spec = pl.GridSpec(
        grid=(B,),
        in_specs=[
            pl.BlockSpec((1, N, in_dim), row),
            pl.BlockSpec((1, N, in_dim), row),
            pl.BlockSpec((1, N, in_dim), row),
            pl.BlockSpec((in_dim, cmp_dim), fixed2),
            pl.BlockSpec((1, cmp_dim), fixed2),
            pl.BlockSpec((in_dim, cmp_dim), fixed2),
            pl.BlockSpec((1, cmp_dim), fixed2),
            pl.BlockSpec((in_dim, out_dim), fixed2),
            pl.BlockSpec((1, out_dim), fixed2),
            pl.BlockSpec((N, N), fixed2),
        ],
        out_specs=pl.BlockSpec((1, N, out_dim), row),
    )
    out = pl.pallas_call(
        functools.partial(_attn_kernel, scale=1.0 / (cmp_dim ** 0.5)),
        grid_spec=grid_spec,
        out_shape=jax.ShapeDtypeStruct((B, N, out_dim), jnp.float32),
    )(xq, xk, xv, Wq, bq.reshape(1, cmp_dim), Wk, bk.reshape(1, cmp_dim),
      Wv, bv.reshape(1, out_dim), mask)
    return out.reshape((B,) + board + (out_dim,))
